# baseline (speedup 1.0000x reference)
import functools

import jax
import jax.numpy as jnp
from jax.experimental import pallas as pl
from jax.experimental.pallas import tpu as pltpu

N = 10000
E = 320000
C = 128
KS = 5
KT = KS * KS
NP = 10240
BN = 512
KC = KT * C


def _mm_kernel(acc_ref, x_ref, w_ref, root_ref, invc_ref, bias_ref, out_ref, st_ref):
    i = pl.program_id(0)
    agg = jnp.dot(acc_ref[...], w_ref[...], preferred_element_type=jnp.float32)
    o = agg * invc_ref[...] + jnp.dot(x_ref[...], root_ref[...],
                                      preferred_element_type=jnp.float32) + bias_ref[...]
    out_ref[...] = o
    rows = i * BN + jax.lax.broadcasted_iota(jnp.int32, (BN, 1), 0)
    om = jnp.where(rows < N, o, 0.0)
    s1 = jnp.sum(om, axis=0, keepdims=True)
    s2 = jnp.sum(om * om, axis=0, keepdims=True)
    st = jnp.concatenate([s1, s2, jnp.zeros((6, s1.shape[1]), jnp.float32)], axis=0)

    @pl.when(i == 0)
    def _():
        st_ref[...] = st

    @pl.when(i > 0)
    def _():
        st_ref[...] += st


def _mm(acc, x, wf, root, invc, bias):
    cout = wf.shape[1]
    grid = NP // BN
    return pl.pallas_call(
        _mm_kernel,
        grid=(grid,),
        in_specs=[
            pl.BlockSpec((BN, KC), lambda i: (i, 0)),
            pl.BlockSpec((BN, C), lambda i: (i, 0)),
            pl.BlockSpec((KC, cout), lambda i: (0, 0)),
            pl.BlockSpec((C, cout), lambda i: (0, 0)),
            pl.BlockSpec((BN, 1), lambda i: (i, 0)),
            pl.BlockSpec((1, cout), lambda i: (0, 0)),
        ],
        out_specs=[
            pl.BlockSpec((BN, cout), lambda i: (i, 0)),
            pl.BlockSpec((8, cout), lambda i: (0, 0)),
        ],
        out_shape=[
            jax.ShapeDtypeStruct((NP, cout), jnp.float32),
            jax.ShapeDtypeStruct((8, cout), jnp.float32),
        ],
    )(acc, x, wf, root, invc, bias)


def _bn_kernel(o_ref, st_ref, g_ref, b_ref, x_ref):
    mu = st_ref[0:1, :] / N
    var = st_ref[1:2, :] / N - mu * mu
    x_ref[...] = jnp.maximum(
        (o_ref[...] - mu) * jax.lax.rsqrt(var + 1e-5) * g_ref[...] + b_ref[...], 0.0)


def _bn_relu(o, st, g, b):
    cout = o.shape[1]
    grid = NP // BN
    return pl.pallas_call(
        _bn_kernel,
        grid=(grid,),
        in_specs=[
            pl.BlockSpec((BN, cout), lambda i: (i, 0)),
            pl.BlockSpec((8, cout), lambda i: (0, 0)),
            pl.BlockSpec((1, cout), lambda i: (0, 0)),
            pl.BlockSpec((1, cout), lambda i: (0, 0)),
        ],
        out_specs=pl.BlockSpec((BN, cout), lambda i: (i, 0)),
        out_shape=jax.ShapeDtypeStruct((NP, cout), jnp.float32),
    )(o, st, g, b)


def _spline_meta(edge_attr):
    ea = edge_attr[:, :2]
    v = ea * (KS - 1)
    k0f = jnp.floor(v)
    frac = v - k0f
    k0 = k0f.astype(jnp.int32)
    kx0 = jnp.clip(k0[:, 0], 0, KS - 1)
    ky0 = jnp.clip(k0[:, 1], 0, KS - 1)
    kx1 = jnp.clip(k0[:, 0] + 1, 0, KS - 1)
    ky1 = jnp.clip(k0[:, 1] + 1, 0, KS - 1)
    fx = frac[:, 0]
    fy = frac[:, 1]
    wi = jnp.stack([kx0 + KS * ky0, kx1 + KS * ky0, kx0 + KS * ky1, kx1 + KS * ky1], 1)
    basis = jnp.stack([(1 - fx) * (1 - fy), fx * (1 - fy), (1 - fx) * fy, fx * fy], 1)
    return basis, wi


def _acc_jnp(x, src, dst, basis, wi):
    upd = basis[:, :, None] * x[src][:, None, :]
    acc = jnp.zeros((NP * KT, C), jnp.float32).at[dst[:, None] * KT + wi].add(upd)
    return acc.reshape(NP, KC)


def kernel(x, edge_index, edge_attr, W1, root1, g1, b1, W2, root2, g2, b2,
           W3, root3, g3, b3, Wr, rr, br, Wc, rc, bc, Wo, ro, bo):
    src = edge_index[0]
    dst = edge_index[1]
    basis, wi = _spline_meta(edge_attr)

    cnt = jnp.zeros((NP,), jnp.float32).at[dst].add(1.0)
    invc = (1.0 / jnp.maximum(cnt, 1.0))[:, None]

    xp = jnp.pad(x, ((0, NP - N), (0, 0)))
    zb = jnp.zeros((1, C), jnp.float32)

    def conv(xin, wflat, root, bias):
        acc = _acc_jnp(xin, src, dst, basis, wi)
        return _mm(acc, xin, wflat, root, invc, bias)

    w1f = W1.reshape(KC, C)
    o1, st1 = conv(xp, w1f, root1, zb)
    x1 = _bn_relu(o1, st1, g1[None, :], b1[None, :])

    w23 = jnp.concatenate([W2.reshape(KC, C), W3.reshape(KC, C)], axis=1)
    r23 = jnp.concatenate([root2, root3], axis=1)
    o23, st23 = conv(x1, w23, r23, jnp.zeros((1, 2 * C), jnp.float32))
    g23 = jnp.concatenate([g2, g3])[None, :]
    b23 = jnp.concatenate([b2, b3])[None, :]
    x23 = _bn_relu(o23, st23, g23, b23)
    x2 = x23[:, :C]
    x3 = x23[:, C:]

    wrf = Wr.reshape(KC, 4)
    oreg, _ = conv(x2, wrf, rr, br[None, :])

    wco = jnp.concatenate([Wc.reshape(KC, 101), Wo.reshape(KC, 1)], axis=1)
    rco = jnp.concatenate([rc, ro], axis=1)
    bco = jnp.concatenate([bc, bo])[None, :]
    oco, _ = conv(x3, wco, rco, bco)

    cls = oco[:N, :101]
    obj = oco[:N, 101:102]
    reg = oreg[:N, :]
    return (cls, reg, obj)


# speedup vs baseline: 4.2640x; 4.2640x over previous
import functools

import jax
import jax.numpy as jnp
from jax import lax
from jax.experimental import pallas as pl
from jax.experimental.pallas import tpu as pltpu
from jax.experimental.pallas import tpu_sc as plsc

N = 10000
E = 320000
C = 128
KS = 5
KT = KS * KS
NP = 10240
BN = 512
KC = KT * C

SUBW = 32
NBK = NP // SUBW
NWORK = 32
PW = NBK // NWORK
CH = 64
NCH = E // CH
ACCW = SUBW * KC


def _sc_acc_kernel(x_hbm, meta_hbm, boff_hbm, acc_hbm,
                   acc_v, srcb, valsb, xb, boff_v,
                   gsem0, gsem1, msem0, msem1):
    cid = lax.axis_index("c")
    sid = lax.axis_index("s")
    w = sid * 2 + cid
    pltpu.sync_copy(boff_hbm, boff_v)
    iota = lax.iota(jnp.int32, 16)
    nchm1 = NCH - 1

    def subblock(sb, carry):
        B = w * PW + sb
        bv = boff_v[pl.ds(B * 16, 16)]
        start = bv[0]
        end = bv[1]

        c0 = start // CH
        c1 = (end + CH - 1) // CH
        dbase = B * SUBW

        cc0 = jnp.minimum(c0, nchm1)
        ccn = jnp.minimum(c0 + 1, nchm1)
        pltpu.async_copy(meta_hbm.at[cc0], valsb.at[pl.ds(0, 4 * CH)], msem0)
        pltpu.async_copy(meta_hbm.at[ccn], valsb.at[pl.ds(4 * CH, 4 * CH)],
                         msem1)

        def zbody(z, c2):
            acc_v[pl.ds(z * 16, 16)] = jnp.zeros((16,), jnp.float32)
            return c2
        lax.fori_loop(0, ACCW // 16, zbody, 0, unroll=8)

        pltpu.make_async_copy(meta_hbm.at[cc0], valsb.at[pl.ds(0, 4 * CH)],
                              msem0).wait()
        for g in range(CH // 16):
            srcb[pl.ds(g * 16, 16)] = valsb[pl.ds(g * 16, 16)].astype(jnp.int32)
        pltpu.async_copy(x_hbm.at[srcb.at[pl.ds(0, CH)]], xb.at[pl.ds(0, CH), :],
                         gsem0)
        pltpu.make_async_copy(meta_hbm.at[ccn], valsb.at[pl.ds(4 * CH, 4 * CH)],
                              msem1).wait()


        def chunk(ci, c3):
            par = (ci - c0) & 1
            cc2 = jnp.minimum(ci + 2, nchm1)
            eb = ci * CH

            @pl.when(par == 0)
            def _():
                pltpu.make_async_copy(x_hbm.at[srcb.at[pl.ds(0, CH)]],
                                      xb.at[pl.ds(0, CH), :], gsem0).wait()

            @pl.when(par == 1)
            def _():
                pltpu.make_async_copy(x_hbm.at[srcb.at[pl.ds(CH, CH)]],
                                      xb.at[pl.ds(CH, CH), :], gsem1).wait()

            nbase = (1 - par) * (4 * CH)
            for g in range(CH // 16):
                srcb[pl.ds((1 - par) * CH + g * 16, 16)] = (
                    valsb[pl.ds(nbase + g * 16, 16)].astype(jnp.int32))

            @pl.when(par == 0)
            def _():
                pltpu.async_copy(x_hbm.at[srcb.at[pl.ds(CH, CH)]],
                                 xb.at[pl.ds(CH, CH), :], gsem1)

            @pl.when(par == 1)
            def _():
                pltpu.async_copy(x_hbm.at[srcb.at[pl.ds(0, CH)]],
                                 xb.at[pl.ds(0, CH), :], gsem0)

            vbase = par * (4 * CH)
            dsts = [valsb[pl.ds(vbase + CH + g * 16, 16)]
                    for g in range(CH // 16)]
            fxs = [valsb[pl.ds(vbase + 2 * CH + g * 16, 16)]
                   for g in range(CH // 16)]
            fys = [valsb[pl.ds(vbase + 3 * CH + g * 16, 16)]
                   for g in range(CH // 16)]

            @pl.when(par == 0)
            def _():
                pltpu.async_copy(meta_hbm.at[cc2], valsb.at[pl.ds(0, 4 * CH)],
                                 msem0)

            @pl.when(par == 1)
            def _():
                pltpu.async_copy(meta_hbm.at[cc2],
                                 valsb.at[pl.ds(4 * CH, 4 * CH)], msem1)

            rbase = par * CH
            for g in range(CH // 16):
                off = g * 16
                dstv = dsts[g].astype(jnp.int32)
                fxv = fxs[g]
                fyv = fys[g]
                eid = eb + off + iota
                valid = (eid >= start) & (eid < end)
                dl = jnp.clip(dstv - dbase, 0, SUBW - 1)
                vx = fxv * (KS - 1.0)
                vy = fyv * (KS - 1.0)
                kx = jnp.clip(vx.astype(jnp.int32), 0, KS - 1)
                ky = jnp.clip(vy.astype(jnp.int32), 0, KS - 1)
                fracx = vx - kx.astype(jnp.float32)
                fracy = vy - ky.astype(jnp.float32)
                kxp = jnp.minimum(kx + 1, KS - 1)
                kyp = jnp.minimum(ky + 1, KS - 1)
                vf = jnp.where(valid, 1.0, 0.0).astype(jnp.float32)
                wx0 = 1.0 - fracx
                wy0 = 1.0 - fracy
                w00 = wx0 * wy0 * vf
                w10 = fracx * wy0 * vf
                w01 = wx0 * fracy * vf
                w11 = fracx * fracy * vf
                base = dl * KC
                A0 = base + (kx + KS * ky) * C
                dmix = (kxp - kx) + 2 * (kyp - ky)
                for i in range(16):
                    row = rbase + off + i
                    a0 = A0[i]
                    d = dmix[i]
                    a1 = a0 + (d & 1) * C
                    a2 = a0 + (d >> 1) * (KS * C)
                    a3 = a2 + (d & 1) * C
                    lane = jnp.full((16,), i, jnp.int32)
                    wb = (jnp.take(w00, lane), jnp.take(w10, lane),
                          jnp.take(w01, lane), jnp.take(w11, lane))
                    xr = [xb[row, pl.ds(16 * j, 16)] for j in range(8)]
                    for a_s, w_s in zip((a0, a1, a2, a3), wb):
                        for j in range(8):
                            plsc.addupdate(acc_v.at[pl.ds(a_s + 16 * j, 16)],
                                           w_s * xr[j])

            @pl.when(par == 0)
            def _():
                pltpu.make_async_copy(meta_hbm.at[cc2],
                                      valsb.at[pl.ds(0, 4 * CH)], msem0).wait()

            @pl.when(par == 1)
            def _():
                pltpu.make_async_copy(meta_hbm.at[cc2],
                                      valsb.at[pl.ds(4 * CH, 4 * CH)],
                                      msem1).wait()
            return c3
        lax.fori_loop(c0, c1, chunk, 0)

        par_end = (c1 - c0) & 1

        @pl.when(par_end == 0)
        def _():
            pltpu.make_async_copy(x_hbm.at[srcb.at[pl.ds(0, CH)]],
                                  xb.at[pl.ds(0, CH), :], gsem0).wait()

        @pl.when(par_end == 1)
        def _():
            pltpu.make_async_copy(x_hbm.at[srcb.at[pl.ds(CH, CH)]],
                                  xb.at[pl.ds(CH, CH), :], gsem1).wait()

        pltpu.sync_copy(acc_v, acc_hbm.at[pl.ds(B * ACCW, ACCW)])
        return carry
    lax.fori_loop(0, PW, subblock, 0)


def _sc_acc(xp, meta_r, boff):
    mesh = plsc.VectorSubcoreMesh(core_axis_name="c", subcore_axis_name="s")
    f = functools.partial(
        pl.kernel,
        out_type=jax.ShapeDtypeStruct((NP * KC,), jnp.float32),
        mesh=mesh,
        scratch_types=[
            pltpu.VMEM((ACCW,), jnp.float32),
            pltpu.VMEM((2 * CH,), jnp.int32),
            pltpu.VMEM((8 * CH,), jnp.float32),
            pltpu.VMEM((2 * CH, C), jnp.float32),
            pltpu.VMEM((NBK * 16,), jnp.int32),
            pltpu.SemaphoreType.DMA,
            pltpu.SemaphoreType.DMA,
            pltpu.SemaphoreType.DMA,
            pltpu.SemaphoreType.DMA,
        ],
    )(_sc_acc_kernel)
    return f(xp, meta_r, boff).reshape(NP, KC)


def _mm_kernel(acc_ref, x_ref, w_ref, root_ref, invc_ref, bias_ref, out_ref, st_ref):
    i = pl.program_id(0)
    agg = jnp.dot(acc_ref[...], w_ref[...], preferred_element_type=jnp.float32)
    o = agg * invc_ref[...] + jnp.dot(x_ref[...], root_ref[...],
                                      preferred_element_type=jnp.float32) + bias_ref[...]
    out_ref[...] = o
    rows = i * BN + jax.lax.broadcasted_iota(jnp.int32, (BN, 1), 0)
    om = jnp.where(rows < N, o, 0.0)
    s1 = jnp.sum(om, axis=0, keepdims=True)
    s2 = jnp.sum(om * om, axis=0, keepdims=True)
    st = jnp.concatenate([s1, s2, jnp.zeros((6, s1.shape[1]), jnp.float32)], axis=0)

    @pl.when(i == 0)
    def _():
        st_ref[...] = st

    @pl.when(i > 0)
    def _():
        st_ref[...] += st


def _mm(acc, x, wf, root, invc, bias):
    cout = wf.shape[1]
    grid = NP // BN
    return pl.pallas_call(
        _mm_kernel,
        grid=(grid,),
        in_specs=[
            pl.BlockSpec((BN, KC), lambda i: (i, 0)),
            pl.BlockSpec((BN, C), lambda i: (i, 0)),
            pl.BlockSpec((KC, cout), lambda i: (0, 0)),
            pl.BlockSpec((C, cout), lambda i: (0, 0)),
            pl.BlockSpec((BN, 1), lambda i: (i, 0)),
            pl.BlockSpec((1, cout), lambda i: (0, 0)),
        ],
        out_specs=[
            pl.BlockSpec((BN, cout), lambda i: (i, 0)),
            pl.BlockSpec((8, cout), lambda i: (0, 0)),
        ],
        out_shape=[
            jax.ShapeDtypeStruct((NP, cout), jnp.float32),
            jax.ShapeDtypeStruct((8, cout), jnp.float32),
        ],
    )(acc, x, wf, root, invc, bias)


def _bn_kernel(o_ref, st_ref, g_ref, b_ref, x_ref):
    mu = st_ref[0:1, :] / N
    var = st_ref[1:2, :] / N - mu * mu
    x_ref[...] = jnp.maximum(
        (o_ref[...] - mu) * jax.lax.rsqrt(var + 1e-5) * g_ref[...] + b_ref[...], 0.0)


def _bn_relu(o, st, g, b):
    cout = o.shape[1]
    grid = NP // BN
    return pl.pallas_call(
        _bn_kernel,
        grid=(grid,),
        in_specs=[
            pl.BlockSpec((BN, cout), lambda i: (i, 0)),
            pl.BlockSpec((8, cout), lambda i: (0, 0)),
            pl.BlockSpec((1, cout), lambda i: (0, 0)),
            pl.BlockSpec((1, cout), lambda i: (0, 0)),
        ],
        out_specs=pl.BlockSpec((BN, cout), lambda i: (i, 0)),
        out_shape=jax.ShapeDtypeStruct((NP, cout), jnp.float32),
    )(o, st, g, b)


def kernel(x, edge_index, edge_attr, W1, root1, g1, b1, W2, root2, g2, b2,
           W3, root3, g3, b3, Wr, rr, br, Wc, rc, bc, Wo, ro, bo):
    src = edge_index[0]
    dst = edge_index[1]

    perm = jnp.argsort(dst)
    src_s = jnp.take(src, perm)
    dst_s = jnp.take(dst, perm)
    ea_s = jnp.take(edge_attr[:, :2], perm, axis=0)
    meta_r = jnp.stack([src_s.astype(jnp.float32), dst_s.astype(jnp.float32),
                        ea_s[:, 0], ea_s[:, 1]],
                       0).reshape(4, NCH, CH).transpose(1, 0, 2).reshape(NCH, 4 * CH)
    bounds = jnp.arange(0, NP + 1, SUBW, dtype=jnp.int32)
    boff0 = jnp.searchsorted(dst_s, bounds).astype(jnp.int32)
    boff = jnp.zeros((NBK, 16), jnp.int32)
    boff = boff.at[:, 0].set(boff0[:-1]).at[:, 1].set(boff0[1:]).reshape(-1)

    node_off = jnp.searchsorted(dst_s, jnp.arange(NP + 1, dtype=jnp.int32))
    cnt = jnp.diff(node_off).astype(jnp.float32)
    invc = (1.0 / jnp.maximum(cnt, 1.0))[:, None]

    xp = jnp.pad(x, ((0, NP - N), (0, 0)))
    zb = jnp.zeros((1, C), jnp.float32)

    def conv(xin, wflat, root, bias):
        acc = _sc_acc(xin, meta_r, boff)
        return _mm(acc, xin, wflat, root, invc, bias)

    w1f = W1.reshape(KC, C)
    o1, st1 = conv(xp, w1f, root1, zb)
    x1 = _bn_relu(o1, st1, g1[None, :], b1[None, :])

    w23 = jnp.concatenate([W2.reshape(KC, C), W3.reshape(KC, C)], axis=1)
    r23 = jnp.concatenate([root2, root3], axis=1)
    o23, st23 = conv(x1, w23, r23, jnp.zeros((1, 2 * C), jnp.float32))
    g23 = jnp.concatenate([g2, g3])[None, :]
    b23 = jnp.concatenate([b2, b3])[None, :]
    x23 = _bn_relu(o23, st23, g23, b23)
    x2 = x23[:, :C]
    x3 = x23[:, C:]

    wrf = Wr.reshape(KC, 4)
    oreg, _ = conv(x2, wrf, rr, br[None, :])

    wco = jnp.concatenate([Wc.reshape(KC, 101), Wo.reshape(KC, 1)], axis=1)
    rco = jnp.concatenate([rc, ro], axis=1)
    bco = jnp.concatenate([bc, bo])[None, :]
    oco, _ = conv(x3, wco, rco, bco)

    cls = oco[:N, :101]
    obj = oco[:N, 101:102]
    reg = oreg[:N, :]
    return (cls, reg, obj)
